# Initial kernel scaffold; baseline (speedup 1.0000x reference)
import jax
import jax.numpy as jnp
from jax.experimental import pallas as pl


def kernel(x, gate_w, gate_b, w1, b1, w2, b2):
    raise NotImplementedError("write your pallas kernel here")


# speedup vs baseline: 1.1492x; 1.1492x over previous
import functools

import jax
import jax.numpy as jnp
from jax import lax
from jax.experimental import pallas as pl
from jax.experimental.pallas import tpu as pltpu
from jax.experimental.pallas import tpu_sc as plsc

D = 1024
E = 8
HH = 4096
S = 2048
NA = 2 * S
TM = 128
NP = 5120
NT = 39
BN = 512
NB = HH // BN
L = 16


def _gate_kernel(x_ref, gw_ref, gb_ref, ei_ref, pw_ref):
    logits = jnp.dot(x_ref[...], gw_ref[...], preferred_element_type=jnp.float32)
    logits = logits + gb_ref[...]
    m1 = jnp.max(logits, axis=-1, keepdims=True)
    i1 = jnp.argmax(logits, axis=-1, keepdims=True)
    lane = jax.lax.broadcasted_iota(jnp.int32, logits.shape, 1)
    masked = jnp.where(lane == i1, -jnp.inf, logits)
    m2 = jnp.max(masked, axis=-1, keepdims=True)
    i2 = jnp.argmax(masked, axis=-1, keepdims=True)
    p1 = 1.0 / (1.0 + jnp.exp(m2 - m1))
    p2 = 1.0 - p1
    ei_ref[...] = jnp.concatenate([i1, i2], axis=1).astype(jnp.int32)
    pw_ref[...] = jnp.concatenate([p1, p2], axis=1)


def _gating(x2, gate_w, gb2):
    return pl.pallas_call(
        _gate_kernel,
        out_shape=(
            jax.ShapeDtypeStruct((S, 2), jnp.int32),
            jax.ShapeDtypeStruct((S, 2), jnp.float32),
        ),
    )(x2, gate_w, gb2)


def _route_kernel(ef_hbm, pos_hbm, tid_hbm, texp_hbm,
                  evm, pos_vm, tid_vm, io_a, io_p, texp_vm, spos, stid):
    c = lax.axis_index("c")
    s = lax.axis_index("s")
    lane = lax.iota(jnp.int32, L)
    zi = jnp.zeros((L,), jnp.int32)

    def _z(i, _):
        tid_vm[pl.ds(i * L, L)] = zi
        io_p[pl.ds(i * L, L)] = lane + i * L
        return 0
    lax.fori_loop(0, NP // L, _z, 0)

    def _z2(i, _):
        pos_vm[pl.ds(i * L, L)] = zi
        io_a[pl.ds(i * L, L)] = lane + i * L
        return 0
    lax.fori_loop(0, NA // L, _z2, 0)

    is_w0 = jnp.logical_and(c == 0, s == 0)

    @pl.when(is_w0)
    def _():
        pltpu.sync_copy(pos_vm, spos)
        pltpu.sync_copy(tid_vm, stid)

    pltpu.sync_copy(ef_hbm, evm)

    def _cnt(i, cv):
        v = evm[pl.ds(i * L, L)]
        for ee in range(E):
            pc = jnp.sum(jnp.where(v == ee, 1, 0))
            cv = cv + jnp.where(lane == ee, pc, 0)
        return cv
    counts = lax.fori_loop(0, NA // L, _cnt, zi)

    padded = ((counts + (TM - 1)) >> 7) << 7
    ends = plsc.cumsum(padded)
    starts = ends - padded

    plsc.subcore_barrier()

    is_router = jnp.logical_and(c == 0, s < E)

    @pl.when(is_router)
    def _():
        e = s
        start_e = jnp.sum(jnp.where(lane == e, starts, 0))

        def _compact(i, off):
            v = evm[pl.ds(i * L, L)]
            m = v == e
            mi = jnp.where(m, 1, 0)
            cum = plsc.cumsum(mi)
            posv = off + cum - 1
            jv = lane + i * L
            plsc.store_scatter(pos_vm, [jv], posv, mask=m)
            tv = lax.shift_right_logical(jv, 1)
            plsc.store_scatter(tid_vm, [posv], tv, mask=m)
            return off + jnp.sum(mi)
        lax.fori_loop(0, NA // L, _compact, start_e)

        pltpu.sync_copy(pos_vm, spos.at[io_a], add=True)
        pltpu.sync_copy(tid_vm, stid.at[io_p], add=True)

    plsc.subcore_barrier()

    @pl.when(is_w0)
    def _():
        for cb in range(3):
            rowv = (lane + cb * L) * TM
            cnt = jnp.zeros((L,), jnp.int32)
            for ee in range(E):
                end_s = jnp.sum(jnp.where(lane == ee, ends, 0))
                cnt = cnt + jnp.where(rowv >= end_s, 1, 0)
            texp_vm[pl.ds(cb * L, L)] = jnp.minimum(cnt, E - 1)
        pltpu.sync_copy(texp_vm, texp_hbm)
        pltpu.sync_copy(spos, pos_hbm)
        pltpu.sync_copy(stid, tid_hbm)


def _routing(eflat):
    mesh = plsc.VectorSubcoreMesh(core_axis_name="c", subcore_axis_name="s")
    f = pl.kernel(
        _route_kernel,
        out_type=(
            jax.ShapeDtypeStruct((NA,), jnp.int32),
            jax.ShapeDtypeStruct((NP,), jnp.int32),
            jax.ShapeDtypeStruct((48,), jnp.int32),
        ),
        mesh=mesh,
        compiler_params=pltpu.CompilerParams(needs_layout_passes=False),
        scratch_types=[
            pltpu.VMEM((NA,), jnp.int32),
            pltpu.VMEM((NA,), jnp.int32),
            pltpu.VMEM((NP,), jnp.int32),
            pltpu.VMEM((NA,), jnp.int32),
            pltpu.VMEM((NP,), jnp.int32),
            pltpu.VMEM((48,), jnp.int32),
            pltpu.VMEM_SHARED((NA,), jnp.int32),
            pltpu.VMEM_SHARED((NP,), jnp.int32),
        ],
    )
    return f(eflat)


RPW = NP // 32
GCH = 40


def _gather_kernel(tid_hbm, x_hbm, xs_hbm, idxv, rows, sem):
    c = lax.axis_index("c")
    s = lax.axis_index("s")
    wid = s * 2 + c
    base = wid * RPW
    pltpu.sync_copy(tid_hbm.at[pl.ds(base, RPW)], idxv)
    for ch in range(RPW // GCH):
        pltpu.async_copy(x_hbm.at[idxv.at[pl.ds(ch * GCH, GCH)]], rows, sem).wait()
        pltpu.sync_copy(rows, xs_hbm.at[pl.ds(base + ch * GCH, GCH)])


def _gather(tid, x2):
    mesh = plsc.VectorSubcoreMesh(core_axis_name="c", subcore_axis_name="s")
    f = pl.kernel(
        _gather_kernel,
        out_type=jax.ShapeDtypeStruct((NP, D), jnp.float32),
        mesh=mesh,
        compiler_params=pltpu.CompilerParams(needs_layout_passes=False),
        scratch_types=[
            pltpu.VMEM((RPW,), jnp.int32),
            pltpu.VMEM((GCH, D), jnp.float32),
            pltpu.SemaphoreType.DMA,
        ],
    )
    return f(tid, x2)


def _gmm_kernel(texp_ref, xs_ref, w1a_ref, w1g_ref, b1a_ref, b1g_ref,
                w2_ref, b2_ref, y_ref):
    n = pl.program_id(0)
    t = pl.program_id(1)
    sl = pl.ds(t * TM, TM)
    xt = xs_ref[sl, :]
    ha = jnp.dot(xt, w1a_ref[0], preferred_element_type=jnp.float32) + b1a_ref[0]
    hg = jnp.dot(xt, w1g_ref[0], preferred_element_type=jnp.float32) + b1g_ref[0]
    act = (hg * jax.nn.sigmoid(hg)) * ha
    part = jnp.dot(act, w2_ref[0], preferred_element_type=jnp.float32)

    @pl.when(n == 0)
    def _():
        y_ref[sl, :] = part + b2_ref[0]

    @pl.when(n > 0)
    def _():
        y_ref[sl, :] += part


def _gmm(texp, xs, w1, b1r, w2, b2r):
    grid_spec = pltpu.PrefetchScalarGridSpec(
        num_scalar_prefetch=1,
        grid=(NB, NT),
        in_specs=[
            pl.BlockSpec((NP, D), lambda n, t, te: (0, 0)),
            pl.BlockSpec((1, D, BN), lambda n, t, te: (te[t], 0, n)),
            pl.BlockSpec((1, D, BN), lambda n, t, te: (te[t], 0, NB + n)),
            pl.BlockSpec((1, 1, BN), lambda n, t, te: (te[t] * 2 * NB + n, 0, 0)),
            pl.BlockSpec((1, 1, BN), lambda n, t, te: (te[t] * 2 * NB + NB + n, 0, 0)),
            pl.BlockSpec((1, BN, D), lambda n, t, te: (te[t], n, 0)),
            pl.BlockSpec((1, 1, D), lambda n, t, te: (te[t], 0, 0)),
        ],
        out_specs=pl.BlockSpec((NP, D), lambda n, t, te: (0, 0)),
    )
    return pl.pallas_call(
        _gmm_kernel,
        grid_spec=grid_spec,
        out_shape=jax.ShapeDtypeStruct((NP, D), jnp.float32),
        compiler_params=pltpu.CompilerParams(
            dimension_semantics=("arbitrary", "arbitrary"),
        ),
    )(texp, xs, w1, w1, b1r, b1r, w2, b2r)


TPW = S // 32
TCH = 8


def _combine_kernel(pos_hbm, pw_hbm, y_hbm, out_hbm, posv, wvv, rows, obuf, sem):
    c = lax.axis_index("c")
    s = lax.axis_index("s")
    lane = lax.iota(jnp.int32, L)
    wid = s * 2 + c
    t0 = wid * TPW
    pltpu.sync_copy(pos_hbm.at[pl.ds(2 * t0, 2 * TPW)], posv)
    pltpu.sync_copy(pw_hbm.at[pl.ds(2 * t0, 2 * TPW)], wvv)
    for ch in range(TPW // TCH):
        pltpu.async_copy(
            y_hbm.at[posv.at[pl.ds(ch * 2 * TCH, 2 * TCH)]], rows, sem
        ).wait()
        wchunk = wvv[pl.ds(ch * 2 * TCH, 2 * TCH)]

        def _tok(i, _):
            w0 = jnp.sum(jnp.where(lane == 2 * i, wchunk, 0.0))
            w1s = jnp.sum(jnp.where(lane == 2 * i + 1, wchunk, 0.0))

            def _col(q, _):
                r0 = rows[2 * i, pl.ds(q * L, L)]
                r1 = rows[2 * i + 1, pl.ds(q * L, L)]
                obuf[i, pl.ds(q * L, L)] = w0 * r0 + w1s * r1
                return 0
            lax.fori_loop(0, D // L, _col, 0)
            return 0
        lax.fori_loop(0, TCH, _tok, 0)
        pltpu.sync_copy(obuf, out_hbm.at[pl.ds(t0 + ch * TCH, TCH)])


def _combine(pos, pwflat, y):
    mesh = plsc.VectorSubcoreMesh(core_axis_name="c", subcore_axis_name="s")
    f = pl.kernel(
        _combine_kernel,
        out_type=jax.ShapeDtypeStruct((S, D), jnp.float32),
        mesh=mesh,
        compiler_params=pltpu.CompilerParams(needs_layout_passes=False),
        scratch_types=[
            pltpu.VMEM((2 * TPW,), jnp.int32),
            pltpu.VMEM((2 * TPW,), jnp.float32),
            pltpu.VMEM((2 * TCH, D), jnp.float32),
            pltpu.VMEM((TCH, D), jnp.float32),
            pltpu.SemaphoreType.DMA,
        ],
    )
    return f(pos, pwflat, y)


@jax.jit
def kernel(x, gate_w, gate_b, w1, b1, w2, b2):
    x2 = x.reshape(S, D)
    gb2 = gate_b.reshape(1, E)
    b1r = b1.reshape(E * 2 * NB, 1, BN)
    b2r = b2.reshape(E, 1, D)

    eidx, pw = _gating(x2, gate_w, gb2)
    eflat = eidx.reshape(NA)
    pwflat = pw.reshape(NA)

    pos, tid, texp = _routing(eflat)
    xs = _gather(tid, x2)
    y = _gmm(texp, xs, w1, b1r, w2, b2r)
    out = _combine(pos, pwflat, y)
    return out.reshape(1, S, D)
